# baseline (speedup 1.0000x reference)
import functools

import jax
import jax.numpy as jnp
from jax import lax
from jax.experimental import pallas as pl
from jax.experimental.pallas import tpu as pltpu
from jax.experimental.pallas import tpu_sc as plsc

B, N, DIM, K = 4, 2048, 256, 16
PPAD = 128
TBLW = DIM + PPAD

K0_BLK = 256


def _k0_body(x_ref, p_ref, wk_ref, wv_ref, t_ref):
    xb = x_ref[0]
    kf = jnp.dot(xb, wk_ref[...], preferred_element_type=jnp.float32)
    v = jnp.dot(xb, wv_ref[...], preferred_element_type=jnp.float32)
    kb = lax.bitcast_convert_type(
        kf.astype(jnp.bfloat16).astype(jnp.float32), jnp.uint32) >> 16
    vb = lax.bitcast_convert_type(
        v.astype(jnp.bfloat16).astype(jnp.float32), jnp.uint32) & jnp.uint32(0xFFFF0000)
    t_ref[0, :, :DIM] = vb | kb
    t_ref[0, :, DIM:] = lax.bitcast_convert_type(p_ref[0], jnp.uint32)


def _build_tables(x, pos128, Wk, Wv):
    return pl.pallas_call(
        _k0_body,
        grid=(B, N // K0_BLK),
        in_specs=[
            pl.BlockSpec((1, K0_BLK, DIM), lambda b, i: (b, i, 0)),
            pl.BlockSpec((1, K0_BLK, PPAD), lambda b, i: (b, i, 0)),
            pl.BlockSpec((DIM, DIM), lambda b, i: (0, 0)),
            pl.BlockSpec((DIM, DIM), lambda b, i: (0, 0)),
        ],
        out_specs=pl.BlockSpec((1, K0_BLK, TBLW), lambda b, i: (b, i, 0)),
        out_shape=jax.ShapeDtypeStruct((B, N, TBLW), jnp.uint32),
    )(x, pos128, Wk, Wv)


K1_BLK = 256


def _k1_body(posq_ref, posall_ref, idx_ref):
    pq = posq_ref[...]
    pa = posall_ref[...]
    sq_q = jnp.sum(pq * pq, axis=1, keepdims=True)
    sq_a = jnp.sum(pa * pa, axis=1, keepdims=True)
    cross = lax.dot_general(pq, pa, (((1,), (1,)), ((), ())),
                            preferred_element_type=jnp.float32)
    d = sq_q + sq_a.T - 2.0 * cross
    lanes = lax.broadcasted_iota(jnp.int32, (K1_BLK, N), 1)
    big = jnp.int32(N)
    inf = jnp.float32(jnp.inf)
    for k in range(K):
        m = jnp.min(d, axis=1, keepdims=True)
        cand = jnp.where(d <= m, lanes, big)
        a = jnp.min(cand, axis=1, keepdims=True)
        idx_ref[:, pl.ds(k, 1)] = a
        d = jnp.where(lanes == a, inf, d)


def _knn_ids_b(pos_b):
    return pl.pallas_call(
        _k1_body,
        grid=(N // K1_BLK,),
        in_specs=[
            pl.BlockSpec((K1_BLK, 16), lambda i: (i, 0)),
            pl.BlockSpec((N, 16), lambda i: (0, 0)),
        ],
        out_specs=pl.BlockSpec((K1_BLK, K), lambda i: (i, 0)),
        out_shape=jax.ShapeDtypeStruct((N, K), jnp.int32),
    )(pos_b, pos_b)


_SC_CHUNK = 128


def _sc_gather_b(tbl, idx):
    info = plsc.get_sparse_core_info()
    nw = info.num_cores * info.num_subcores
    total = N * K
    per_w = total // nw
    n_chunks = per_w // _SC_CHUNK
    mesh = plsc.VectorSubcoreMesh(core_axis_name="c", subcore_axis_name="s")

    @functools.partial(
        pl.kernel,
        mesh=mesh,
        out_type=jax.ShapeDtypeStruct((total, TBLW), jnp.uint32),
        scratch_types=[
            pltpu.VMEM((_SC_CHUNK,), jnp.int32),
            pltpu.VMEM((_SC_CHUNK, TBLW), jnp.uint32),
            pltpu.SemaphoreType.DMA,
        ],
    )
    def k(tbl_hbm, idx_hbm, g_hbm, idx_v, rows_v, sem1):
        wid = lax.axis_index("s") * info.num_cores + lax.axis_index("c")

        def body(c, carry):
            base = wid * per_w + c * _SC_CHUNK
            pltpu.sync_copy(idx_hbm.at[pl.ds(base, _SC_CHUNK)], idx_v)
            pltpu.async_copy(tbl_hbm.at[idx_v], rows_v, sem1).wait()
            pltpu.sync_copy(rows_v, g_hbm.at[pl.ds(base, _SC_CHUNK)])
            return carry

        lax.fori_loop(0, n_chunks, body, 0)

    return k(tbl, idx)


K3_BLK = 256


def _k3_body(x_ref, p_ref, g_ref,
             wq_ref, p1_ref, b1_ref, p2_ref, b2_ref,
             a1_ref, ba1_ref, a2_ref, ba2_ref, wf_ref, bf_ref, out_ref):
    bf = jnp.bfloat16
    xb = x_ref[...]
    pq = p_ref[...]
    g = g_ref[...]
    gk = g[:, :DIM]
    kfv = lax.bitcast_convert_type(gk << 16, jnp.float32)
    vv = lax.bitcast_convert_type(gk & jnp.uint32(0xFFFF0000), jnp.float32)
    xyz = lax.bitcast_convert_type(g[:, DIM:], jnp.float32)

    rel = jnp.broadcast_to(pq[:, None, :], (K3_BLK, K, PPAD)).reshape(K3_BLK * K, PPAD) - xyz
    t1 = jax.nn.relu(jnp.dot(rel.astype(bf), p1_ref[...],
                             preferred_element_type=jnp.float32) + b1_ref[...])
    pe = jnp.dot(t1.astype(bf), p2_ref[...],
                 preferred_element_type=jnp.float32) + b2_ref[...]

    q = jnp.dot(xb.astype(bf), wq_ref[...], preferred_element_type=jnp.float32)
    qrep = jnp.broadcast_to(q[:, None, :], (K3_BLK, K, DIM)).reshape(K3_BLK * K, DIM)
    h = qrep - kfv + pe
    t2 = jax.nn.relu(jnp.dot(h.astype(bf), a1_ref[...],
                             preferred_element_type=jnp.float32) + ba1_ref[...])
    al = jnp.dot(t2.astype(bf), a2_ref[...],
                 preferred_element_type=jnp.float32) + ba2_ref[...]

    al3 = al.reshape(K3_BLK, K, DIM) * (1.0 / 16.0)
    m3 = jnp.max(al3, axis=1, keepdims=True)
    e3 = jnp.exp(al3 - m3)
    s3 = jnp.sum(e3, axis=1, keepdims=True)
    w3 = e3 / s3
    vpe3 = (vv + pe).reshape(K3_BLK, K, DIM)
    agg = jnp.sum(w3 * vpe3, axis=1)

    out_ref[...] = (jnp.dot(agg.astype(bf), wf_ref[...],
                            preferred_element_type=jnp.float32)
                    + bf_ref[...] + xb)


def _fused_block_b(x_b, pos_b, G_b, Wq, P1p, p1, P2, p2, A1, a1, A2, a2, Wf, bias_f):
    wspec = lambda shape: pl.BlockSpec(shape, lambda i: (0, 0))
    return pl.pallas_call(
        _k3_body,
        grid=(N // K3_BLK,),
        in_specs=[
            pl.BlockSpec((K3_BLK, DIM), lambda i: (i, 0)),
            pl.BlockSpec((K3_BLK, PPAD), lambda i: (i, 0)),
            pl.BlockSpec((K3_BLK * K, TBLW), lambda i: (i, 0)),
            wspec((DIM, DIM)),
            wspec((PPAD, DIM)),
            wspec((1, DIM)),
            wspec((DIM, DIM)),
            wspec((1, DIM)),
            wspec((DIM, DIM)),
            wspec((1, DIM)),
            wspec((DIM, DIM)),
            wspec((1, DIM)),
            wspec((DIM, DIM)),
            wspec((1, DIM)),
        ],
        out_specs=pl.BlockSpec((K3_BLK, DIM), lambda i: (i, 0)),
        out_shape=jax.ShapeDtypeStruct((N, DIM), jnp.float32),
    )(x_b, pos_b, G_b, Wq, P1p, p1, P2, p2, A1, a1, A2, a2, Wf, bias_f)


def kernel(x, pos, Wq, Wk, Wv, P1, p1, P2, p2, A1, a1, A2, a2, Wf, bf):
    pos128 = jnp.pad(pos, ((0, 0), (0, 0), (0, PPAD - 3)))
    P1p = jnp.pad(P1, ((0, PPAD - 3), (0, 0)))
    w16 = lambda w: w.astype(jnp.bfloat16)
    b2d = lambda v: v.reshape(1, DIM)

    T = _build_tables(x, pos128, Wk, Wv)
    T2 = T.reshape(B * N, TBLW)
    outs = []
    pos16 = pos128[:, :, :16]
    for b in range(B):
        ids = _knn_ids_b(pos16[b])
        G = _sc_gather_b(T2, ids.reshape(N * K) + b * N)
        outs.append(_fused_block_b(
            x[b], pos128[b], G,
            w16(Wq), w16(P1p), b2d(p1), w16(P2), b2d(p2),
            w16(A1), b2d(a1), w16(A2), b2d(a2), w16(Wf), b2d(bf)))
    return jnp.stack(outs)


# speedup vs baseline: 1.0102x; 1.0102x over previous
import functools

import jax
import jax.numpy as jnp
from jax import lax
from jax.experimental import pallas as pl
from jax.experimental.pallas import tpu as pltpu
from jax.experimental.pallas import tpu_sc as plsc

B, N, DIM, K = 4, 2048, 256, 16
PPAD = 128
TBLW = DIM + PPAD

K0_BLK = 256


def _k0_body(x_ref, p_ref, wk_ref, wv_ref, t_ref):
    xb = x_ref[0]
    kf = jnp.dot(xb, wk_ref[...], preferred_element_type=jnp.float32)
    v = jnp.dot(xb, wv_ref[...], preferred_element_type=jnp.float32)
    kb = lax.bitcast_convert_type(
        kf.astype(jnp.bfloat16).astype(jnp.float32), jnp.uint32) >> 16
    vb = lax.bitcast_convert_type(
        v.astype(jnp.bfloat16).astype(jnp.float32), jnp.uint32) & jnp.uint32(0xFFFF0000)
    t_ref[0, :, :DIM] = vb | kb
    t_ref[0, :, DIM:] = lax.bitcast_convert_type(p_ref[0], jnp.uint32)


def _build_tables(x, pos128, Wk, Wv):
    return pl.pallas_call(
        _k0_body,
        grid=(B, N // K0_BLK),
        in_specs=[
            pl.BlockSpec((1, K0_BLK, DIM), lambda b, i: (b, i, 0)),
            pl.BlockSpec((1, K0_BLK, PPAD), lambda b, i: (b, i, 0)),
            pl.BlockSpec((DIM, DIM), lambda b, i: (0, 0)),
            pl.BlockSpec((DIM, DIM), lambda b, i: (0, 0)),
        ],
        out_specs=pl.BlockSpec((1, K0_BLK, TBLW), lambda b, i: (b, i, 0)),
        out_shape=jax.ShapeDtypeStruct((B, N, TBLW), jnp.uint32),
    )(x, pos128, Wk, Wv)


K1_BLK = 256


def _k1_body(posq_ref, posall_ref, idx_ref):
    pq = posq_ref[...]
    pa = posall_ref[...]
    sq_q = jnp.sum(pq * pq, axis=1, keepdims=True)
    sq_a = jnp.sum(pa * pa, axis=1, keepdims=True)
    cross = lax.dot_general(pq, pa, (((1,), (1,)), ((), ())),
                            preferred_element_type=jnp.float32)
    d = sq_q + sq_a.T - 2.0 * cross
    lanes = lax.broadcasted_iota(jnp.int32, (K1_BLK, N), 1)
    big = jnp.int32(N)
    inf = jnp.float32(jnp.inf)
    for k in range(K):
        m = jnp.min(d, axis=1, keepdims=True)
        cand = jnp.where(d <= m, lanes, big)
        a = jnp.min(cand, axis=1, keepdims=True)
        idx_ref[:, pl.ds(k, 1)] = a
        d = jnp.where(lanes == a, inf, d)


def _knn_ids_b(pos_b):
    return pl.pallas_call(
        _k1_body,
        grid=(N // K1_BLK,),
        in_specs=[
            pl.BlockSpec((K1_BLK, PPAD), lambda i: (i, 0)),
            pl.BlockSpec((N, PPAD), lambda i: (0, 0)),
        ],
        out_specs=pl.BlockSpec((K1_BLK, K), lambda i: (i, 0)),
        out_shape=jax.ShapeDtypeStruct((N, K), jnp.int32),
    )(pos_b, pos_b)


_SC_CHUNK = 128


def _sc_gather_b(tbl, idx):
    info = plsc.get_sparse_core_info()
    nw = info.num_cores * info.num_subcores
    total = N * K
    per_w = total // nw
    n_chunks = per_w // _SC_CHUNK
    mesh = plsc.VectorSubcoreMesh(core_axis_name="c", subcore_axis_name="s")

    @functools.partial(
        pl.kernel,
        mesh=mesh,
        out_type=jax.ShapeDtypeStruct((total, TBLW), jnp.uint32),
        scratch_types=[
            pltpu.VMEM((_SC_CHUNK,), jnp.int32),
            pltpu.VMEM((_SC_CHUNK, TBLW), jnp.uint32),
            pltpu.SemaphoreType.DMA,
        ],
    )
    def k(tbl_hbm, idx_hbm, g_hbm, idx_v, rows_v, sem1):
        wid = lax.axis_index("s") * info.num_cores + lax.axis_index("c")

        def body(c, carry):
            base = wid * per_w + c * _SC_CHUNK
            pltpu.sync_copy(idx_hbm.at[pl.ds(base, _SC_CHUNK)], idx_v)
            pltpu.async_copy(tbl_hbm.at[idx_v], rows_v, sem1).wait()
            pltpu.sync_copy(rows_v, g_hbm.at[pl.ds(base, _SC_CHUNK)])
            return carry

        lax.fori_loop(0, n_chunks, body, 0)

    return k(tbl, idx)


K3_BLK = 256


def _k3_body(x_ref, p_ref, g_ref,
             wq_ref, p1_ref, b1_ref, p2_ref, b2_ref,
             a1_ref, ba1_ref, a2_ref, ba2_ref, wf_ref, bf_ref, out_ref):
    bf = jnp.bfloat16
    xb = x_ref[...]
    pq = p_ref[...]
    g = g_ref[...]
    gk = g[:, :DIM]
    kfv = lax.bitcast_convert_type(gk << 16, jnp.float32)
    vv = lax.bitcast_convert_type(gk & jnp.uint32(0xFFFF0000), jnp.float32)
    xyz = lax.bitcast_convert_type(g[:, DIM:], jnp.float32)

    rel = jnp.broadcast_to(pq[:, None, :], (K3_BLK, K, PPAD)).reshape(K3_BLK * K, PPAD) - xyz
    t1 = jax.nn.relu(jnp.dot(rel.astype(bf), p1_ref[...],
                             preferred_element_type=jnp.float32) + b1_ref[...])
    pe = jnp.dot(t1.astype(bf), p2_ref[...],
                 preferred_element_type=jnp.float32) + b2_ref[...]

    q = jnp.dot(xb.astype(bf), wq_ref[...], preferred_element_type=jnp.float32)
    qrep = jnp.broadcast_to(q[:, None, :], (K3_BLK, K, DIM)).reshape(K3_BLK * K, DIM)
    h = qrep - kfv + pe
    t2 = jax.nn.relu(jnp.dot(h.astype(bf), a1_ref[...],
                             preferred_element_type=jnp.float32) + ba1_ref[...])
    al = jnp.dot(t2.astype(bf), a2_ref[...],
                 preferred_element_type=jnp.float32) + ba2_ref[...]

    al3 = al.reshape(K3_BLK, K, DIM) * (1.0 / 16.0)
    m3 = jnp.max(al3, axis=1, keepdims=True)
    e3 = jnp.exp(al3 - m3)
    s3 = jnp.sum(e3, axis=1, keepdims=True)
    w3 = e3 / s3
    vpe3 = (vv + pe).reshape(K3_BLK, K, DIM)
    agg = jnp.sum(w3 * vpe3, axis=1)

    out_ref[...] = (jnp.dot(agg.astype(bf), wf_ref[...],
                            preferred_element_type=jnp.float32)
                    + bf_ref[...] + xb)


def _fused_block_b(x_b, pos_b, G_b, Wq, P1p, p1, P2, p2, A1, a1, A2, a2, Wf, bias_f):
    wspec = lambda shape: pl.BlockSpec(shape, lambda i: (0, 0))
    return pl.pallas_call(
        _k3_body,
        grid=(N // K3_BLK,),
        in_specs=[
            pl.BlockSpec((K3_BLK, DIM), lambda i: (i, 0)),
            pl.BlockSpec((K3_BLK, PPAD), lambda i: (i, 0)),
            pl.BlockSpec((K3_BLK * K, TBLW), lambda i: (i, 0)),
            wspec((DIM, DIM)),
            wspec((PPAD, DIM)),
            wspec((1, DIM)),
            wspec((DIM, DIM)),
            wspec((1, DIM)),
            wspec((DIM, DIM)),
            wspec((1, DIM)),
            wspec((DIM, DIM)),
            wspec((1, DIM)),
            wspec((DIM, DIM)),
            wspec((1, DIM)),
        ],
        out_specs=pl.BlockSpec((K3_BLK, DIM), lambda i: (i, 0)),
        out_shape=jax.ShapeDtypeStruct((N, DIM), jnp.float32),
    )(x_b, pos_b, G_b, Wq, P1p, p1, P2, p2, A1, a1, A2, a2, Wf, bias_f)


def kernel(x, pos, Wq, Wk, Wv, P1, p1, P2, p2, A1, a1, A2, a2, Wf, bf):
    pos128 = jnp.pad(pos, ((0, 0), (0, 0), (0, PPAD - 3)))
    P1p = jnp.pad(P1, ((0, PPAD - 3), (0, 0)))
    w16 = lambda w: w.astype(jnp.bfloat16)
    b2d = lambda v: v.reshape(1, DIM)

    T = _build_tables(x, pos128, Wk, Wv)
    T2 = T.reshape(B * N, TBLW)
    outs = []
    for b in range(B):
        ids = _knn_ids_b(pos128[b])
        G = _sc_gather_b(T2, ids.reshape(N * K) + b * N)
        outs.append(_fused_block_b(
            x[b], pos128[b], G,
            w16(Wq), w16(P1p), b2d(p1), w16(P2), b2d(p2),
            w16(A1), b2d(a1), w16(A2), b2d(a2), w16(Wf), b2d(bf)))
    return jnp.stack(outs)


# speedup vs baseline: 1.0225x; 1.0122x over previous
import functools

import jax
import jax.numpy as jnp
from jax import lax
from jax.experimental import pallas as pl
from jax.experimental.pallas import tpu as pltpu
from jax.experimental.pallas import tpu_sc as plsc

B, N, DIM, K = 4, 2048, 256, 16
PPAD = 128
TBLW = DIM + PPAD

K0_BLK = 512


def _k0_body(x_ref, p_ref, wk_ref, wv_ref, t_ref):
    xb = x_ref[0]
    kf = jnp.dot(xb, wk_ref[...], preferred_element_type=jnp.float32)
    v = jnp.dot(xb, wv_ref[...], preferred_element_type=jnp.float32)
    kb = lax.bitcast_convert_type(
        kf.astype(jnp.bfloat16).astype(jnp.float32), jnp.uint32) >> 16
    vb = lax.bitcast_convert_type(
        v.astype(jnp.bfloat16).astype(jnp.float32), jnp.uint32) & jnp.uint32(0xFFFF0000)
    t_ref[0, :, :DIM] = vb | kb
    t_ref[0, :, DIM:] = lax.bitcast_convert_type(p_ref[0], jnp.uint32)


def _build_tables(x, pos128, Wk, Wv):
    return pl.pallas_call(
        _k0_body,
        grid=(B, N // K0_BLK),
        in_specs=[
            pl.BlockSpec((1, K0_BLK, DIM), lambda b, i: (b, i, 0)),
            pl.BlockSpec((1, K0_BLK, PPAD), lambda b, i: (b, i, 0)),
            pl.BlockSpec((DIM, DIM), lambda b, i: (0, 0)),
            pl.BlockSpec((DIM, DIM), lambda b, i: (0, 0)),
        ],
        out_specs=pl.BlockSpec((1, K0_BLK, TBLW), lambda b, i: (b, i, 0)),
        out_shape=jax.ShapeDtypeStruct((B, N, TBLW), jnp.uint32),
    )(x, pos128, Wk, Wv)


K1_BLK = 256


def _k1_body(posq_ref, posall_ref, idx_ref):
    pq = posq_ref[...]
    pa = posall_ref[...]
    sq_q = jnp.sum(pq * pq, axis=1, keepdims=True)
    sq_a = jnp.sum(pa * pa, axis=1, keepdims=True)
    cross = lax.dot_general(pq, pa, (((1,), (1,)), ((), ())),
                            preferred_element_type=jnp.float32)
    d = sq_q + sq_a.T - 2.0 * cross
    lanes = lax.broadcasted_iota(jnp.int32, (K1_BLK, N), 1)
    big = jnp.int32(N)
    inf = jnp.float32(jnp.inf)
    for k in range(K):
        m = jnp.min(d, axis=1, keepdims=True)
        cand = jnp.where(d <= m, lanes, big)
        a = jnp.min(cand, axis=1, keepdims=True)
        idx_ref[:, pl.ds(k, 1)] = a
        d = jnp.where(lanes == a, inf, d)


def _knn_ids_b(pos_b):
    return pl.pallas_call(
        _k1_body,
        grid=(N // K1_BLK,),
        in_specs=[
            pl.BlockSpec((K1_BLK, PPAD), lambda i: (i, 0)),
            pl.BlockSpec((N, PPAD), lambda i: (0, 0)),
        ],
        out_specs=pl.BlockSpec((K1_BLK, K), lambda i: (i, 0)),
        out_shape=jax.ShapeDtypeStruct((N, K), jnp.int32),
    )(pos_b, pos_b)


_SC_CHUNK = 128


def _sc_gather_b(tbl, idx):
    info = plsc.get_sparse_core_info()
    nw = info.num_cores * info.num_subcores
    total = N * K
    per_w = total // nw
    n_chunks = per_w // _SC_CHUNK
    mesh = plsc.VectorSubcoreMesh(core_axis_name="c", subcore_axis_name="s")

    @functools.partial(
        pl.kernel,
        mesh=mesh,
        out_type=jax.ShapeDtypeStruct((total, TBLW), jnp.uint32),
        scratch_types=[
            pltpu.VMEM((_SC_CHUNK,), jnp.int32),
            pltpu.VMEM((_SC_CHUNK, TBLW), jnp.uint32),
            pltpu.SemaphoreType.DMA,
        ],
    )
    def k(tbl_hbm, idx_hbm, g_hbm, idx_v, rows_v, sem1):
        wid = lax.axis_index("s") * info.num_cores + lax.axis_index("c")

        def body(c, carry):
            base = wid * per_w + c * _SC_CHUNK
            pltpu.sync_copy(idx_hbm.at[pl.ds(base, _SC_CHUNK)], idx_v)
            pltpu.async_copy(tbl_hbm.at[idx_v], rows_v, sem1).wait()
            pltpu.sync_copy(rows_v, g_hbm.at[pl.ds(base, _SC_CHUNK)])
            return carry

        lax.fori_loop(0, n_chunks, body, 0)

    return k(tbl, idx)


K3_BLK = 256


def _k3_body(x_ref, p_ref, g_ref,
             wq_ref, p1_ref, b1_ref, p2_ref, b2_ref,
             a1_ref, ba1_ref, a2_ref, ba2_ref, wf_ref, bf_ref, out_ref):
    bf = jnp.bfloat16
    xb = x_ref[...]
    pq = p_ref[...]
    g = g_ref[...]
    gk = g[:, :DIM]
    kfv = lax.bitcast_convert_type(gk << 16, jnp.float32)
    vv = lax.bitcast_convert_type(gk & jnp.uint32(0xFFFF0000), jnp.float32)
    xyz = lax.bitcast_convert_type(g[:, DIM:], jnp.float32)

    rel = jnp.broadcast_to(pq[:, None, :], (K3_BLK, K, PPAD)).reshape(K3_BLK * K, PPAD) - xyz
    t1 = jax.nn.relu(jnp.dot(rel.astype(bf), p1_ref[...],
                             preferred_element_type=jnp.float32) + b1_ref[...])
    pe = jnp.dot(t1.astype(bf), p2_ref[...],
                 preferred_element_type=jnp.float32) + b2_ref[...]

    q = jnp.dot(xb.astype(bf), wq_ref[...], preferred_element_type=jnp.float32)
    qrep = jnp.broadcast_to(q[:, None, :], (K3_BLK, K, DIM)).reshape(K3_BLK * K, DIM)
    h = qrep - kfv + pe
    t2 = jax.nn.relu(jnp.dot(h.astype(bf), a1_ref[...],
                             preferred_element_type=jnp.float32) + ba1_ref[...])
    al = jnp.dot(t2.astype(bf), a2_ref[...],
                 preferred_element_type=jnp.float32) + ba2_ref[...]

    al3 = al.reshape(K3_BLK, K, DIM) * (1.0 / 16.0)
    m3 = jnp.max(al3, axis=1, keepdims=True)
    e3 = jnp.exp(al3 - m3)
    s3 = jnp.sum(e3, axis=1, keepdims=True)
    w3 = e3 / s3
    vpe3 = (vv + pe).reshape(K3_BLK, K, DIM)
    agg = jnp.sum(w3 * vpe3, axis=1)

    out_ref[...] = (jnp.dot(agg.astype(bf), wf_ref[...],
                            preferred_element_type=jnp.float32)
                    + bf_ref[...] + xb)


def _fused_block_b(x_b, pos_b, G_b, Wq, P1p, p1, P2, p2, A1, a1, A2, a2, Wf, bias_f):
    wspec = lambda shape: pl.BlockSpec(shape, lambda i: (0, 0))
    return pl.pallas_call(
        _k3_body,
        grid=(N // K3_BLK,),
        in_specs=[
            pl.BlockSpec((K3_BLK, DIM), lambda i: (i, 0)),
            pl.BlockSpec((K3_BLK, PPAD), lambda i: (i, 0)),
            pl.BlockSpec((K3_BLK * K, TBLW), lambda i: (i, 0)),
            wspec((DIM, DIM)),
            wspec((PPAD, DIM)),
            wspec((1, DIM)),
            wspec((DIM, DIM)),
            wspec((1, DIM)),
            wspec((DIM, DIM)),
            wspec((1, DIM)),
            wspec((DIM, DIM)),
            wspec((1, DIM)),
            wspec((DIM, DIM)),
            wspec((1, DIM)),
        ],
        out_specs=pl.BlockSpec((K3_BLK, DIM), lambda i: (i, 0)),
        out_shape=jax.ShapeDtypeStruct((N, DIM), jnp.float32),
    )(x_b, pos_b, G_b, Wq, P1p, p1, P2, p2, A1, a1, A2, a2, Wf, bias_f)


def kernel(x, pos, Wq, Wk, Wv, P1, p1, P2, p2, A1, a1, A2, a2, Wf, bf):
    pos128 = jnp.pad(pos, ((0, 0), (0, 0), (0, PPAD - 3)))
    P1p = jnp.pad(P1, ((0, PPAD - 3), (0, 0)))
    w16 = lambda w: w.astype(jnp.bfloat16)
    b2d = lambda v: v.reshape(1, DIM)

    T = _build_tables(x, pos128, Wk, Wv)
    T2 = T.reshape(B * N, TBLW)
    outs = []
    for b in range(B):
        ids = _knn_ids_b(pos128[b])
        G = _sc_gather_b(T2, ids.reshape(N * K) + b * N)
        outs.append(_fused_block_b(
            x[b], pos128[b], G,
            w16(Wq), w16(P1p), b2d(p1), w16(P2), b2d(p2),
            w16(A1), b2d(a1), w16(A2), b2d(a2), w16(Wf), b2d(bf)))
    return jnp.stack(outs)


# speedup vs baseline: 1.0340x; 1.0112x over previous
import functools

import jax
import jax.numpy as jnp
from jax import lax
from jax.experimental import pallas as pl
from jax.experimental.pallas import tpu as pltpu
from jax.experimental.pallas import tpu_sc as plsc

B, N, DIM, K = 4, 2048, 256, 16
PPAD = 128
TBLW = DIM + PPAD

K0_BLK = 1024


def _k0_body(x_ref, p_ref, wk_ref, wv_ref, t_ref):
    xb = x_ref[0]
    kf = jnp.dot(xb, wk_ref[...], preferred_element_type=jnp.float32)
    v = jnp.dot(xb, wv_ref[...], preferred_element_type=jnp.float32)
    kb = lax.bitcast_convert_type(
        kf.astype(jnp.bfloat16).astype(jnp.float32), jnp.uint32) >> 16
    vb = lax.bitcast_convert_type(
        v.astype(jnp.bfloat16).astype(jnp.float32), jnp.uint32) & jnp.uint32(0xFFFF0000)
    t_ref[0, :, :DIM] = vb | kb
    t_ref[0, :, DIM:] = lax.bitcast_convert_type(p_ref[0], jnp.uint32)


def _build_tables(x, pos128, Wk, Wv):
    return pl.pallas_call(
        _k0_body,
        grid=(B, N // K0_BLK),
        in_specs=[
            pl.BlockSpec((1, K0_BLK, DIM), lambda b, i: (b, i, 0)),
            pl.BlockSpec((1, K0_BLK, PPAD), lambda b, i: (b, i, 0)),
            pl.BlockSpec((DIM, DIM), lambda b, i: (0, 0)),
            pl.BlockSpec((DIM, DIM), lambda b, i: (0, 0)),
        ],
        out_specs=pl.BlockSpec((1, K0_BLK, TBLW), lambda b, i: (b, i, 0)),
        out_shape=jax.ShapeDtypeStruct((B, N, TBLW), jnp.uint32),
    )(x, pos128, Wk, Wv)


K1_BLK = 256


def _k1_body(posq_ref, posall_ref, idx_ref):
    pq = posq_ref[...]
    pa = posall_ref[...]
    sq_q = jnp.sum(pq * pq, axis=1, keepdims=True)
    sq_a = jnp.sum(pa * pa, axis=1, keepdims=True)
    cross = lax.dot_general(pq, pa, (((1,), (1,)), ((), ())),
                            preferred_element_type=jnp.float32)
    d = sq_q + sq_a.T - 2.0 * cross
    lanes = lax.broadcasted_iota(jnp.int32, (K1_BLK, N), 1)
    big = jnp.int32(N)
    inf = jnp.float32(jnp.inf)
    for k in range(K):
        m = jnp.min(d, axis=1, keepdims=True)
        cand = jnp.where(d <= m, lanes, big)
        a = jnp.min(cand, axis=1, keepdims=True)
        idx_ref[:, pl.ds(k, 1)] = a
        d = jnp.where(lanes == a, inf, d)


def _knn_ids_b(pos_b):
    return pl.pallas_call(
        _k1_body,
        grid=(N // K1_BLK,),
        in_specs=[
            pl.BlockSpec((K1_BLK, PPAD), lambda i: (i, 0)),
            pl.BlockSpec((N, PPAD), lambda i: (0, 0)),
        ],
        out_specs=pl.BlockSpec((K1_BLK, K), lambda i: (i, 0)),
        out_shape=jax.ShapeDtypeStruct((N, K), jnp.int32),
    )(pos_b, pos_b)


_SC_CHUNK = 128


def _sc_gather_b(tbl, idx):
    info = plsc.get_sparse_core_info()
    nw = info.num_cores * info.num_subcores
    total = N * K
    per_w = total // nw
    n_chunks = per_w // _SC_CHUNK
    mesh = plsc.VectorSubcoreMesh(core_axis_name="c", subcore_axis_name="s")

    @functools.partial(
        pl.kernel,
        mesh=mesh,
        out_type=jax.ShapeDtypeStruct((total, TBLW), jnp.uint32),
        scratch_types=[
            pltpu.VMEM((_SC_CHUNK,), jnp.int32),
            pltpu.VMEM((_SC_CHUNK, TBLW), jnp.uint32),
            pltpu.SemaphoreType.DMA,
        ],
    )
    def k(tbl_hbm, idx_hbm, g_hbm, idx_v, rows_v, sem1):
        wid = lax.axis_index("s") * info.num_cores + lax.axis_index("c")

        def body(c, carry):
            base = wid * per_w + c * _SC_CHUNK
            pltpu.sync_copy(idx_hbm.at[pl.ds(base, _SC_CHUNK)], idx_v)
            pltpu.async_copy(tbl_hbm.at[idx_v], rows_v, sem1).wait()
            pltpu.sync_copy(rows_v, g_hbm.at[pl.ds(base, _SC_CHUNK)])
            return carry

        lax.fori_loop(0, n_chunks, body, 0)

    return k(tbl, idx)


K3_BLK = 256


def _k3_body(x_ref, p_ref, g_ref,
             wq_ref, p1_ref, b1_ref, p2_ref, b2_ref,
             a1_ref, ba1_ref, a2_ref, ba2_ref, wf_ref, bf_ref, out_ref):
    bf = jnp.bfloat16
    xb = x_ref[...]
    pq = p_ref[...]
    g = g_ref[...]
    gk = g[:, :DIM]
    kfv = lax.bitcast_convert_type(gk << 16, jnp.float32)
    vv = lax.bitcast_convert_type(gk & jnp.uint32(0xFFFF0000), jnp.float32)
    xyz = lax.bitcast_convert_type(g[:, DIM:], jnp.float32)

    rel = jnp.broadcast_to(pq[:, None, :], (K3_BLK, K, PPAD)).reshape(K3_BLK * K, PPAD) - xyz
    t1 = jax.nn.relu(jnp.dot(rel.astype(bf), p1_ref[...],
                             preferred_element_type=jnp.float32) + b1_ref[...])
    pe = jnp.dot(t1.astype(bf), p2_ref[...],
                 preferred_element_type=jnp.float32) + b2_ref[...]

    q = jnp.dot(xb.astype(bf), wq_ref[...], preferred_element_type=jnp.float32)
    qrep = jnp.broadcast_to(q[:, None, :], (K3_BLK, K, DIM)).reshape(K3_BLK * K, DIM)
    h = qrep - kfv + pe
    t2 = jax.nn.relu(jnp.dot(h.astype(bf), a1_ref[...],
                             preferred_element_type=jnp.float32) + ba1_ref[...])
    al = jnp.dot(t2.astype(bf), a2_ref[...],
                 preferred_element_type=jnp.float32) + ba2_ref[...]

    al3 = al.reshape(K3_BLK, K, DIM) * (1.0 / 16.0)
    m3 = jnp.max(al3, axis=1, keepdims=True)
    e3 = jnp.exp(al3 - m3)
    s3 = jnp.sum(e3, axis=1, keepdims=True)
    w3 = e3 / s3
    vpe3 = (vv + pe).reshape(K3_BLK, K, DIM)
    agg = jnp.sum(w3 * vpe3, axis=1)

    out_ref[...] = (jnp.dot(agg.astype(bf), wf_ref[...],
                            preferred_element_type=jnp.float32)
                    + bf_ref[...] + xb)


def _fused_block_b(x_b, pos_b, G_b, Wq, P1p, p1, P2, p2, A1, a1, A2, a2, Wf, bias_f):
    wspec = lambda shape: pl.BlockSpec(shape, lambda i: (0, 0))
    return pl.pallas_call(
        _k3_body,
        grid=(N // K3_BLK,),
        in_specs=[
            pl.BlockSpec((K3_BLK, DIM), lambda i: (i, 0)),
            pl.BlockSpec((K3_BLK, PPAD), lambda i: (i, 0)),
            pl.BlockSpec((K3_BLK * K, TBLW), lambda i: (i, 0)),
            wspec((DIM, DIM)),
            wspec((PPAD, DIM)),
            wspec((1, DIM)),
            wspec((DIM, DIM)),
            wspec((1, DIM)),
            wspec((DIM, DIM)),
            wspec((1, DIM)),
            wspec((DIM, DIM)),
            wspec((1, DIM)),
            wspec((DIM, DIM)),
            wspec((1, DIM)),
        ],
        out_specs=pl.BlockSpec((K3_BLK, DIM), lambda i: (i, 0)),
        out_shape=jax.ShapeDtypeStruct((N, DIM), jnp.float32),
    )(x_b, pos_b, G_b, Wq, P1p, p1, P2, p2, A1, a1, A2, a2, Wf, bias_f)


def kernel(x, pos, Wq, Wk, Wv, P1, p1, P2, p2, A1, a1, A2, a2, Wf, bf):
    pos128 = jnp.pad(pos, ((0, 0), (0, 0), (0, PPAD - 3)))
    P1p = jnp.pad(P1, ((0, PPAD - 3), (0, 0)))
    w16 = lambda w: w.astype(jnp.bfloat16)
    b2d = lambda v: v.reshape(1, DIM)

    T = _build_tables(x, pos128, Wk, Wv)
    T2 = T.reshape(B * N, TBLW)
    outs = []
    for b in range(B):
        ids = _knn_ids_b(pos128[b])
        G = _sc_gather_b(T2, ids.reshape(N * K) + b * N)
        outs.append(_fused_block_b(
            x[b], pos128[b], G,
            w16(Wq), w16(P1p), b2d(p1), w16(P2), b2d(p2),
            w16(A1), b2d(a1), w16(A2), b2d(a2), w16(Wf), b2d(bf)))
    return jnp.stack(outs)


# speedup vs baseline: 1.0346x; 1.0006x over previous
import functools

import jax
import jax.numpy as jnp
from jax import lax
from jax.experimental import pallas as pl
from jax.experimental.pallas import tpu as pltpu
from jax.experimental.pallas import tpu_sc as plsc

B, N, DIM, K = 4, 2048, 256, 16
PPAD = 128
TBLW = DIM + PPAD

K0_BLK = 2048


def _k0_body(x_ref, p_ref, wk_ref, wv_ref, t_ref):
    xb = x_ref[0]
    kf = jnp.dot(xb, wk_ref[...], preferred_element_type=jnp.float32)
    v = jnp.dot(xb, wv_ref[...], preferred_element_type=jnp.float32)
    kb = lax.bitcast_convert_type(
        kf.astype(jnp.bfloat16).astype(jnp.float32), jnp.uint32) >> 16
    vb = lax.bitcast_convert_type(
        v.astype(jnp.bfloat16).astype(jnp.float32), jnp.uint32) & jnp.uint32(0xFFFF0000)
    t_ref[0, :, :DIM] = vb | kb
    t_ref[0, :, DIM:] = lax.bitcast_convert_type(p_ref[0], jnp.uint32)


def _build_tables(x, pos128, Wk, Wv):
    return pl.pallas_call(
        _k0_body,
        grid=(B, N // K0_BLK),
        in_specs=[
            pl.BlockSpec((1, K0_BLK, DIM), lambda b, i: (b, i, 0)),
            pl.BlockSpec((1, K0_BLK, PPAD), lambda b, i: (b, i, 0)),
            pl.BlockSpec((DIM, DIM), lambda b, i: (0, 0)),
            pl.BlockSpec((DIM, DIM), lambda b, i: (0, 0)),
        ],
        out_specs=pl.BlockSpec((1, K0_BLK, TBLW), lambda b, i: (b, i, 0)),
        out_shape=jax.ShapeDtypeStruct((B, N, TBLW), jnp.uint32),
    )(x, pos128, Wk, Wv)


K1_BLK = 256


def _k1_body(posq_ref, posall_ref, idx_ref):
    pq = posq_ref[...]
    pa = posall_ref[...]
    sq_q = jnp.sum(pq * pq, axis=1, keepdims=True)
    sq_a = jnp.sum(pa * pa, axis=1, keepdims=True)
    cross = lax.dot_general(pq, pa, (((1,), (1,)), ((), ())),
                            preferred_element_type=jnp.float32)
    d = sq_q + sq_a.T - 2.0 * cross
    lanes = lax.broadcasted_iota(jnp.int32, (K1_BLK, N), 1)
    big = jnp.int32(N)
    inf = jnp.float32(jnp.inf)
    for k in range(K):
        m = jnp.min(d, axis=1, keepdims=True)
        cand = jnp.where(d <= m, lanes, big)
        a = jnp.min(cand, axis=1, keepdims=True)
        idx_ref[:, pl.ds(k, 1)] = a
        d = jnp.where(lanes == a, inf, d)


def _knn_ids_b(pos_b):
    return pl.pallas_call(
        _k1_body,
        grid=(N // K1_BLK,),
        in_specs=[
            pl.BlockSpec((K1_BLK, PPAD), lambda i: (i, 0)),
            pl.BlockSpec((N, PPAD), lambda i: (0, 0)),
        ],
        out_specs=pl.BlockSpec((K1_BLK, K), lambda i: (i, 0)),
        out_shape=jax.ShapeDtypeStruct((N, K), jnp.int32),
    )(pos_b, pos_b)


_SC_CHUNK = 128


def _sc_gather_b(tbl, idx):
    info = plsc.get_sparse_core_info()
    nw = info.num_cores * info.num_subcores
    total = N * K
    per_w = total // nw
    n_chunks = per_w // _SC_CHUNK
    mesh = plsc.VectorSubcoreMesh(core_axis_name="c", subcore_axis_name="s")

    @functools.partial(
        pl.kernel,
        mesh=mesh,
        out_type=jax.ShapeDtypeStruct((total, TBLW), jnp.uint32),
        scratch_types=[
            pltpu.VMEM((_SC_CHUNK,), jnp.int32),
            pltpu.VMEM((_SC_CHUNK, TBLW), jnp.uint32),
            pltpu.SemaphoreType.DMA,
        ],
    )
    def k(tbl_hbm, idx_hbm, g_hbm, idx_v, rows_v, sem1):
        wid = lax.axis_index("s") * info.num_cores + lax.axis_index("c")

        def body(c, carry):
            base = wid * per_w + c * _SC_CHUNK
            pltpu.sync_copy(idx_hbm.at[pl.ds(base, _SC_CHUNK)], idx_v)
            pltpu.async_copy(tbl_hbm.at[idx_v], rows_v, sem1).wait()
            pltpu.sync_copy(rows_v, g_hbm.at[pl.ds(base, _SC_CHUNK)])
            return carry

        lax.fori_loop(0, n_chunks, body, 0)

    return k(tbl, idx)


K3_BLK = 256


def _k3_body(x_ref, p_ref, g_ref,
             wq_ref, p1_ref, b1_ref, p2_ref, b2_ref,
             a1_ref, ba1_ref, a2_ref, ba2_ref, wf_ref, bf_ref, out_ref):
    bf = jnp.bfloat16
    xb = x_ref[...]
    pq = p_ref[...]
    g = g_ref[...]
    gk = g[:, :DIM]
    kfv = lax.bitcast_convert_type(gk << 16, jnp.float32)
    vv = lax.bitcast_convert_type(gk & jnp.uint32(0xFFFF0000), jnp.float32)
    xyz = lax.bitcast_convert_type(g[:, DIM:], jnp.float32)

    rel = jnp.broadcast_to(pq[:, None, :], (K3_BLK, K, PPAD)).reshape(K3_BLK * K, PPAD) - xyz
    t1 = jax.nn.relu(jnp.dot(rel.astype(bf), p1_ref[...],
                             preferred_element_type=jnp.float32) + b1_ref[...])
    pe = jnp.dot(t1.astype(bf), p2_ref[...],
                 preferred_element_type=jnp.float32) + b2_ref[...]

    q = jnp.dot(xb.astype(bf), wq_ref[...], preferred_element_type=jnp.float32)
    qrep = jnp.broadcast_to(q[:, None, :], (K3_BLK, K, DIM)).reshape(K3_BLK * K, DIM)
    h = qrep - kfv + pe
    t2 = jax.nn.relu(jnp.dot(h.astype(bf), a1_ref[...],
                             preferred_element_type=jnp.float32) + ba1_ref[...])
    al = jnp.dot(t2.astype(bf), a2_ref[...],
                 preferred_element_type=jnp.float32) + ba2_ref[...]

    al3 = al.reshape(K3_BLK, K, DIM) * (1.0 / 16.0)
    m3 = jnp.max(al3, axis=1, keepdims=True)
    e3 = jnp.exp(al3 - m3)
    s3 = jnp.sum(e3, axis=1, keepdims=True)
    w3 = e3 / s3
    vpe3 = (vv + pe).reshape(K3_BLK, K, DIM)
    agg = jnp.sum(w3 * vpe3, axis=1)

    out_ref[...] = (jnp.dot(agg.astype(bf), wf_ref[...],
                            preferred_element_type=jnp.float32)
                    + bf_ref[...] + xb)


def _fused_block_b(x_b, pos_b, G_b, Wq, P1p, p1, P2, p2, A1, a1, A2, a2, Wf, bias_f):
    wspec = lambda shape: pl.BlockSpec(shape, lambda i: (0, 0))
    return pl.pallas_call(
        _k3_body,
        grid=(N // K3_BLK,),
        in_specs=[
            pl.BlockSpec((K3_BLK, DIM), lambda i: (i, 0)),
            pl.BlockSpec((K3_BLK, PPAD), lambda i: (i, 0)),
            pl.BlockSpec((K3_BLK * K, TBLW), lambda i: (i, 0)),
            wspec((DIM, DIM)),
            wspec((PPAD, DIM)),
            wspec((1, DIM)),
            wspec((DIM, DIM)),
            wspec((1, DIM)),
            wspec((DIM, DIM)),
            wspec((1, DIM)),
            wspec((DIM, DIM)),
            wspec((1, DIM)),
            wspec((DIM, DIM)),
            wspec((1, DIM)),
        ],
        out_specs=pl.BlockSpec((K3_BLK, DIM), lambda i: (i, 0)),
        out_shape=jax.ShapeDtypeStruct((N, DIM), jnp.float32),
    )(x_b, pos_b, G_b, Wq, P1p, p1, P2, p2, A1, a1, A2, a2, Wf, bias_f)


def kernel(x, pos, Wq, Wk, Wv, P1, p1, P2, p2, A1, a1, A2, a2, Wf, bf):
    pos128 = jnp.pad(pos, ((0, 0), (0, 0), (0, PPAD - 3)))
    P1p = jnp.pad(P1, ((0, PPAD - 3), (0, 0)))
    w16 = lambda w: w.astype(jnp.bfloat16)
    b2d = lambda v: v.reshape(1, DIM)

    T = _build_tables(x, pos128, Wk, Wv)
    T2 = T.reshape(B * N, TBLW)
    outs = []
    for b in range(B):
        ids = _knn_ids_b(pos128[b])
        G = _sc_gather_b(T2, ids.reshape(N * K) + b * N)
        outs.append(_fused_block_b(
            x[b], pos128[b], G,
            w16(Wq), w16(P1p), b2d(p1), w16(P2), b2d(p2),
            w16(A1), b2d(a1), w16(A2), b2d(a2), w16(Wf), b2d(bf)))
    return jnp.stack(outs)
